# baseline (speedup 1.0000x reference)
import functools

import jax
import jax.numpy as jnp
from jax.experimental import pallas as pl
from jax.experimental.pallas import tpu as pltpu

_EPS = 1e-5
_WOFF = 8


def _in_relu(x):
    axes = tuple(range(x.ndim - 1))
    mu = jnp.mean(x, axis=axes, keepdims=True)
    var = jnp.mean(jnp.square(x - mu), axis=axes, keepdims=True)
    return jnp.maximum((x - mu) * jax.lax.rsqrt(var + _EPS), 0.0)


def _taps(hp_ref, H, W, C, stride):
    Ho, Wo = H // stride, W // stride
    taps = []
    for kh in range(3):
        for kw in range(3):
            if stride == 1:
                p = hp_ref[kh:kh + Ho, _WOFF - 1 + kw:_WOFF - 1 + kw + Wo, :]
            else:
                p = hp_ref[pl.ds(kh, Ho, stride=stride),
                           pl.ds(_WOFF - 1 + kw, Wo, stride=stride), :]
            taps.append(p.reshape(Ho * Wo, C).astype(jnp.bfloat16))
    return taps


def _block_kernel(x_ref, w1_ref, wsc_ref, w2_ref, out_ref, hp1_ref, hp2_ref,
                  *, H, W, Cin, Cout, stride):
    Ho, Wo = H // stride, W // stride
    f32 = jnp.float32

    h = _in_relu(x_ref[0].astype(f32))
    hp1_ref[...] = jnp.zeros_like(hp1_ref)
    hp1_ref[1:H + 1, _WOFF:_WOFF + W, :] = h.astype(hp1_ref.dtype)

    taps1 = _taps(hp1_ref, H, W, Cin, stride)
    cols1 = jnp.concatenate(taps1, axis=-1)
    out1 = jnp.dot(cols1, w1_ref[...], preferred_element_type=f32)

    sc_t = jax.lax.dot_general(wsc_ref[...], taps1[4],
                               (((0,), (1,)), ((), ())),
                               preferred_element_type=f32)

    h2 = _in_relu(out1)
    hp2_ref[...] = jnp.zeros_like(hp2_ref)
    hp2_ref[1:Ho + 1, _WOFF:_WOFF + Wo, :] = h2.reshape(Ho, Wo, Cout).astype(hp2_ref.dtype)

    taps2 = _taps(hp2_ref, Ho, Wo, Cout, 1)
    cols2 = jnp.concatenate(taps2, axis=-1)
    out_t = jax.lax.dot_general(w2_ref[...], cols2,
                                (((0,), (1,)), ((), ())),
                                preferred_element_type=f32)
    out_ref[0] = (out_t + sc_t).astype(out_ref.dtype)


def kernel(x, w1, w2, w_sc):
    stride, scaler_rate = 2, 0.5
    scale = 1.0 / scaler_rate
    N, Cin, H, W = x.shape
    Cout = w1.shape[0]
    Ho, Wo = H // stride, W // stride
    bf16 = jnp.bfloat16

    w1_mat = (jnp.transpose(w1, (2, 3, 1, 0)).reshape(9 * Cin, Cout) * scale).astype(bf16)
    w2_mat = (jnp.transpose(w2, (2, 3, 1, 0)).reshape(9 * Cout, Cout) * scale).astype(bf16)
    wsc_mat = (jnp.transpose(w_sc[:, :, 0, 0], (1, 0)) * scale).astype(bf16)

    xh = jnp.transpose(x.astype(bf16), (0, 2, 3, 1))

    kfn = functools.partial(_block_kernel, H=H, W=W, Cin=Cin, Cout=Cout, stride=stride)
    out_t = pl.pallas_call(
        kfn,
        grid=(N,),
        in_specs=[
            pl.BlockSpec((1, H, W, Cin), lambda n: (n, 0, 0, 0)),
            pl.BlockSpec((9 * Cin, Cout), lambda n: (0, 0)),
            pl.BlockSpec((Cin, Cout), lambda n: (0, 0)),
            pl.BlockSpec((9 * Cout, Cout), lambda n: (0, 0)),
        ],
        out_specs=pl.BlockSpec((1, Cout, Ho * Wo), lambda n: (n, 0, 0)),
        out_shape=jax.ShapeDtypeStruct((N, Cout, Ho * Wo), x.dtype),
        scratch_shapes=[
            pltpu.VMEM((H + 2, _WOFF + W + 8, Cin), jnp.float32),
            pltpu.VMEM((Ho + 2, _WOFF + Wo + 8, Cout), bf16),
        ],
        compiler_params=pltpu.CompilerParams(
            dimension_semantics=("parallel",),
            vmem_limit_bytes=100 * 1024 * 1024,
        ),
    )(xh, w1_mat, wsc_mat, w2_mat)

    return out_t.reshape(N, Cout, Ho, Wo)
